# Initial kernel scaffold; baseline (speedup 1.0000x reference)
import jax
import jax.numpy as jnp
from jax.experimental import pallas as pl


def kernel(ent_embeds, rel_embeds, global_emb_list, w_bases1, w_comp1, loop_w1, w_bases2, w_comp2, loop_w2, edge_index, edge_type, node_ids_graph, s_len, s_tem, r_tem, reverse):
    raise NotImplementedError("write your pallas kernel here")


# speedup vs baseline: 1.7265x; 1.7265x over previous
import functools

import jax
import jax.numpy as jnp
from jax import lax
from jax.experimental import pallas as pl
from jax.experimental.pallas import tpu as pltpu
from jax.experimental.pallas import tpu_sc as plsc

H = 128
NB = 16
NREL = 230
N = 10000
E = 40000
B = 2048
SEQ = 10
TT = B * SEQ

NC = 2
NS = 16
NW = NC * NS
L = 16

EP = 40960
EPW = EP // NW
K = 16
NCHUNK = EPW // K
NPAD = 10240
HALF = NPAD // 2
ACC_ROWS = 5632
GARB = 5500
ZR = ACC_ROWS // NS // 8
DUMP_PT = HALF // NS

MM_RB = 1000


def _mm_body(x_ref, w_ref, o_ref):
    o_ref[...] = jnp.dot(x_ref[...], w_ref[...],
                         preferred_element_type=jnp.float32)


def _matmul(x, w):
    n, h = x.shape
    m = w.shape[1]
    grid = n // MM_RB
    return pl.pallas_call(
        _mm_body,
        grid=(grid,),
        in_specs=[pl.BlockSpec((MM_RB, h), lambda i: (i, 0)),
                  pl.BlockSpec((h, m), lambda i: (0, 0))],
        out_specs=pl.BlockSpec((MM_RB, m), lambda i: (i, 0)),
        out_shape=jax.ShapeDtypeStruct((n, m), jnp.float32),
    )(x, w)


def _norm_mm_body(agg_ref, deg_ref, self_ref, w_ref, o_ref, *, act):
    agg = agg_ref[...].sum(axis=0)
    deg = deg_ref[...].sum(axis=0)[:, 0:1]
    h = agg / jnp.maximum(deg, 1.0) + self_ref[...]
    if act:
        h = jnp.maximum(h, 0.0)
    o_ref[...] = jnp.dot(h, w_ref[...], preferred_element_type=jnp.float32)


def _norm_matmul(agg, deg, xbcat, w, act):
    m = w.shape[1]
    grid = N // MM_RB
    return pl.pallas_call(
        functools.partial(_norm_mm_body, act=act),
        grid=(grid,),
        in_specs=[pl.BlockSpec((NC, MM_RB, H), lambda i: (0, i, 0)),
                  pl.BlockSpec((NC, MM_RB, H), lambda i: (0, i, 0)),
                  pl.BlockSpec((MM_RB, H), lambda i: (i, NB)),
                  pl.BlockSpec((H, m), lambda i: (0, 0))],
        out_specs=pl.BlockSpec((MM_RB, m), lambda i: (i, 0)),
        out_shape=jax.ShapeDtypeStruct((N, m), jnp.float32),
    )(agg, deg, xbcat, w)


def _norm_body(agg_ref, deg_ref, self_ref, o_ref):
    agg = agg_ref[...].sum(axis=0)
    deg = deg_ref[...].sum(axis=0)[:, 0:1]
    o_ref[...] = agg / jnp.maximum(deg, 1.0) + self_ref[...]


def _norm(agg, deg, xbcat):
    grid = N // MM_RB
    return pl.pallas_call(
        _norm_body,
        grid=(grid,),
        in_specs=[pl.BlockSpec((NC, MM_RB, H), lambda i: (0, i, 0)),
                  pl.BlockSpec((NC, MM_RB, H), lambda i: (0, i, 0)),
                  pl.BlockSpec((MM_RB, H), lambda i: (i, NB))],
        out_specs=pl.BlockSpec((MM_RB, H), lambda i: (i, 0)),
        out_shape=jax.ShapeDtypeStruct((N, H), jnp.float32),
    )(agg, deg, xbcat)


def _vtake(x, idx):
    dn = lax.GatherDimensionNumbers(offset_dims=(), collapsed_slice_dims=(0,),
                                    start_index_map=(0,))
    return lax.gather(x, idx[:, None], dn, slice_sizes=(1,),
                      mode=lax.GatherScatterMode.PROMISE_IN_BOUNDS)


def _edge_body(with_deg, xb_hbm, wcomp_hbm, src_hbm, dstl_hbm, dstu_hbm,
               et_hbm, zrow_hbm, ones_hbm,
               agg_out, deg_out, msg_out,
               coef_v, src_v, dstl_v, dstu_v, et_v, rows_v, msg_v, ones_v,
               acc_sh, sem, sem2):
    c = lax.axis_index("c")
    s = lax.axis_index("s")
    w = s * NC + c

    def zero_acc():
        for k in range(8):
            pltpu.sync_copy(zrow_hbm,
                            acc_sh.at[pl.ds(s * (8 * ZR) + k * ZR, ZR)])

    def dump_acc(out, half):
        pltpu.sync_copy(acc_sh.at[pl.ds(s * DUMP_PT, DUMP_PT)],
                        out.at[c, pl.ds(half * HALF + s * DUMP_PT, DUMP_PT)])

    pltpu.sync_copy(src_hbm.at[pl.ds(w * EPW, EPW)], src_v)
    pltpu.sync_copy(dstl_hbm.at[pl.ds(w * NCHUNK, NCHUNK)], dstl_v)
    pltpu.sync_copy(dstu_hbm.at[pl.ds(w * NCHUNK, NCHUNK)], dstu_v)
    pltpu.sync_copy(et_hbm.at[pl.ds(w * EPW, EPW)], et_v)

    if with_deg:
        pltpu.sync_copy(ones_hbm, ones_v)
        for half, dv in ((0, dstl_v), (1, dstu_v)):
            zero_acc()
            plsc.subcore_barrier()

            def dchunk(j, _, dv=dv):
                pltpu.sync_copy(ones_v, acc_sh.at[dv.at[j]], add=True)
                return ()

            lax.fori_loop(0, NCHUNK, dchunk, ())
            plsc.subcore_barrier()
            dump_acc(deg_out, half)
            plsc.subcore_barrier()

    zero_acc()
    plsc.subcore_barrier()

    def chunk(j, _):
        cdma = pltpu.async_copy(wcomp_hbm.at[et_v.at[pl.ds(j * K, K)]],
                                coef_v, sem2)
        pltpu.async_copy(xb_hbm.at[src_v.at[pl.ds(j * K, K)]], rows_v, sem).wait()
        cdma.wait()
        for e in range(K):
            coef = coef_v[e, pl.ds(0, NB)]
            acc = [jnp.zeros((L,), jnp.float32) for _ in range(H // L)]
            for b in range(NB):
                cb = _vtake(coef, jnp.full((L,), b, jnp.int32))
                for d in range(H // L):
                    acc[d] = acc[d] + cb * rows_v[e, pl.ds(b * H + d * L, L)]
            for d in range(H // L):
                msg_v[e, pl.ds(d * L, L)] = acc[d]
        pltpu.sync_copy(msg_v, msg_out.at[pl.ds(w * EPW + j * K, K)])
        pltpu.sync_copy(msg_v, acc_sh.at[dstl_v.at[j]], add=True)
        return ()

    lax.fori_loop(0, NCHUNK, chunk, ())
    plsc.subcore_barrier()
    dump_acc(agg_out, 0)
    plsc.subcore_barrier()

    zero_acc()
    plsc.subcore_barrier()

    def uchunk(j, _):
        pltpu.sync_copy(msg_out.at[pl.ds(w * EPW + j * K, K)], msg_v)
        pltpu.sync_copy(msg_v, acc_sh.at[dstu_v.at[j]], add=True)
        return ()

    lax.fori_loop(0, NCHUNK, uchunk, ())
    plsc.subcore_barrier()
    dump_acc(agg_out, 1)


def _edge_pass(xb, wcomp, srcp, dstl2, dstu2, etp, zrow, ones, with_deg):
    mesh = plsc.VectorSubcoreMesh(core_axis_name="c", subcore_axis_name="s",
                                  num_cores=NC, num_subcores=NS)
    out_type = (jax.ShapeDtypeStruct((NC, NPAD, H), jnp.float32),
                jax.ShapeDtypeStruct((NC, NPAD, H), jnp.float32),
                jax.ShapeDtypeStruct((EP, H), jnp.float32))
    kern = pl.kernel(
        functools.partial(_edge_body, with_deg),
        out_type=out_type,
        mesh=mesh,
        scratch_types=[
            pltpu.VMEM((K, H), jnp.float32),
            pltpu.VMEM((EPW,), jnp.int32),
            pltpu.VMEM((NCHUNK, K), jnp.int32),
            pltpu.VMEM((NCHUNK, K), jnp.int32),
            pltpu.VMEM((EPW,), jnp.int32),
            pltpu.VMEM((K, NB * H), jnp.float32),
            pltpu.VMEM((K, H), jnp.float32),
            pltpu.VMEM((K, H), jnp.float32),
            pltpu.VMEM_SHARED((ACC_ROWS, H), jnp.float32),
            pltpu.SemaphoreType.DMA,
            pltpu.SemaphoreType.DMA,
        ],
    )
    agg, deg, _ = kern(xb, wcomp, srcp, dstl2, dstu2, etp, zrow, ones)
    return agg, deg


def _gather_body(h2_hbm, ent_hbm, rel_hbm, ig_hbm, ie_hbm, ir_hbm,
                 g_out, e_out, r_out, idx_v, stage_v, sem):
    c = lax.axis_index("c")
    s = lax.axis_index("s")
    w = s * NC + c
    tpw = TT // NW
    nb = tpw // H
    for tab, idx, out in ((h2_hbm, ig_hbm, g_out),
                          (ent_hbm, ie_hbm, e_out),
                          (rel_hbm, ir_hbm, r_out)):
        pltpu.sync_copy(idx.at[w], idx_v)
        for k in range(nb):
            pltpu.async_copy(tab.at[idx_v.at[k]], stage_v, sem).wait()
            pltpu.sync_copy(stage_v, out.at[pl.ds(w * tpw + k * H, H)])


def _token_gather(h2, ent, rel, ig, ie, ir):
    mesh = plsc.VectorSubcoreMesh(core_axis_name="c", subcore_axis_name="s",
                                  num_cores=NC, num_subcores=NS)
    out_type = (jax.ShapeDtypeStruct((TT, H), jnp.float32),
                jax.ShapeDtypeStruct((TT, H), jnp.float32),
                jax.ShapeDtypeStruct((TT, H), jnp.float32))
    kern = pl.kernel(
        _gather_body,
        out_type=out_type,
        mesh=mesh,
        scratch_types=[
            pltpu.VMEM((TT // NW // H, H), jnp.int32),
            pltpu.VMEM((H, H), jnp.float32),
            pltpu.SemaphoreType.DMA,
        ],
    )
    return kern(h2, ent, rel, ig, ie, ir)


def _asm_body(g_ref, e_ref, r_ref, gl_ref, t_ref, tr_ref):
    g = g_ref[...]
    en = e_ref[...]
    re = r_ref[...]
    gl = gl_ref[...]
    t_ref[...] = jnp.concatenate([g, en, re, gl], axis=1)
    tr_ref[...] = jnp.concatenate([g, en, gl], axis=1)


def _assemble(g_tok, e_tok, r_tok, glob):
    blk = 640
    grid = TT // blk
    return pl.pallas_call(
        _asm_body,
        grid=(grid,),
        in_specs=[pl.BlockSpec((blk, H), lambda i: (i, 0))] * 4,
        out_specs=[pl.BlockSpec((blk, 4 * H), lambda i: (i, 0)),
                   pl.BlockSpec((blk, 3 * H), lambda i: (i, 0))],
        out_shape=[jax.ShapeDtypeStruct((TT, 4 * H), jnp.float32),
                   jax.ShapeDtypeStruct((TT, 3 * H), jnp.float32)],
    )(g_tok, e_tok, r_tok, glob)


def kernel(ent_embeds, rel_embeds, global_emb_list, w_bases1, w_comp1, loop_w1,
           w_bases2, w_comp2, loop_w2, edge_index, edge_type, node_ids_graph,
           s_len, s_tem, r_tem, reverse):
    i32 = jnp.int32
    src = edge_index[0].astype(i32)
    dst = edge_index[1].astype(i32)
    et = (edge_type + reverse * NREL).astype(i32)

    pad = EP - E
    srcp = jnp.concatenate([src, jnp.zeros((pad,), i32)])
    dstp = jnp.concatenate([dst, jnp.full((pad,), N, i32)])
    etp = jnp.concatenate([et, jnp.zeros((pad,), i32)])
    dstl2 = jnp.where(dstp < HALF, dstp, GARB).reshape(EP // K, K)
    dstu2 = jnp.where(dstp >= HALF, dstp - HALF, GARB).reshape(EP // K, K)

    wcat1 = jnp.concatenate(
        [w_bases1.transpose(1, 0, 2).reshape(H, NB * H), loop_w1], axis=1)
    wcat2 = jnp.concatenate(
        [w_bases2.transpose(1, 0, 2).reshape(H, NB * H), loop_w2], axis=1)

    zrow = jnp.zeros((ZR, H), jnp.float32)
    ones = jnp.ones((K, H), jnp.float32)
    wcomp1p = jnp.pad(w_comp1, ((0, 0), (0, H - NB)))
    wcomp2p = jnp.pad(w_comp2, ((0, 0), (0, H - NB)))

    xbcat1 = _matmul(ent_embeds, wcat1)
    xb1 = xbcat1[:, :NB * H]
    agg1, deg = _edge_pass(xb1, wcomp1p, srcp, dstl2, dstu2, etp,
                           zrow, ones, True)
    xbcat2 = _norm_matmul(agg1, deg, xbcat1, wcat2, True)
    xb2 = xbcat2[:, :NB * H]
    agg2, _ = _edge_pass(xb2, wcomp2p, srcp, dstl2, dstu2, etp,
                         zrow, ones, False)
    h2 = _norm(agg2, deg, xbcat2)

    ig = node_ids_graph.astype(i32).reshape(NW, TT // NW // H, H)
    ie = jnp.broadcast_to(s_tem.astype(i32)[:, None],
                          (B, SEQ)).reshape(NW, TT // NW // H, H)
    ir = jnp.broadcast_to(r_tem.astype(i32)[:, None],
                          (B, SEQ)).reshape(NW, TT // NW // H, H)
    g_tok, e_tok, r_tok = _token_gather(h2, ent_embeds, rel_embeds, ig, ie, ir)
    t2d, tr2d = _assemble(g_tok, e_tok, r_tok, global_emb_list)
    return (t2d.reshape(B, SEQ, 4 * H), tr2d.reshape(B, SEQ, 3 * H))
